# baseline (speedup 1.0000x reference)
import functools

import jax
import jax.numpy as jnp
from jax import lax
from jax.experimental import pallas as pl
from jax.experimental.pallas import tpu as pltpu
from jax.experimental.pallas import tpu_sc as plsc

E = 64
K = 2
CAP = 512
DIN = 1024
DHID = 1024
DOUT = 1024
NTOK = 8192
TBLK = 1024
NB = NTOK // TBLK
SLOTS = E * CAP
XG_ROWS = SLOTS + CAP

NW = 32
TOK_PER_W = NTOK // NW
CHUNK = 32
NCHUNK = TOK_PER_W // CHUNK


def _router_body(x_ref, wr_ref, br_ref,
                 ss0_ref, ss1_ref, sg0_ref, sg1_ref, g0_ref, g1_ref,
                 xbf_ref, counts_ref, aux_ref, psum_ref):
    b = pl.program_id(0)

    @pl.when(b == 0)
    def _():
        counts_ref[...] = jnp.zeros_like(counts_ref)
        psum_ref[...] = jnp.zeros_like(psum_ref)

    x = x_ref[...]
    xbf_ref[...] = x.astype(jnp.bfloat16)
    logits = jnp.dot(x, wr_ref[...], preferred_element_type=jnp.float32)
    logits = logits + br_ref[...]
    m = jnp.max(logits, axis=1, keepdims=True)
    ex = jnp.exp(logits - m)
    probs = ex / jnp.sum(ex, axis=1, keepdims=True)
    psum_ref[...] += jnp.sum(probs, axis=0, keepdims=True)

    lane = lax.broadcasted_iota(jnp.int32, (TBLK, E), 1)
    m1 = jnp.max(probs, axis=1, keepdims=True)
    idx1 = jnp.min(jnp.where(probs == m1, lane, E), axis=1, keepdims=True)
    pm = jnp.where(lane == idx1, -1.0, probs)
    m2 = jnp.max(pm, axis=1, keepdims=True)
    idx2 = jnp.min(jnp.where(pm == m2, lane, E), axis=1, keepdims=True)

    oh = ((lane == idx1) | (lane == idx2)).astype(jnp.float32)
    base = counts_ref[...]
    r = lax.broadcasted_iota(jnp.int32, (TBLK, TBLK), 0)
    c = lax.broadcasted_iota(jnp.int32, (TBLK, TBLK), 1)
    tri = (c < r).astype(jnp.float32)
    csum_ex = jnp.dot(tri, oh, preferred_element_type=jnp.float32)
    posf = csum_ex + base
    counts_ref[...] = base + jnp.sum(oh, axis=0, keepdims=True)

    for idxk, mk, ss_ref, sg_ref, g_ref in (
            (idx1, m1, ss0_ref, sg0_ref, g0_ref),
            (idx2, m2, ss1_ref, sg1_ref, g1_ref)):
        pk = jnp.sum(jnp.where(lane == idxk, posf, 0.0),
                     axis=1, keepdims=True).astype(jnp.int32)
        slotv = idxk * CAP + pk
        valid = pk < CAP
        ss_ref[0, 0, :] = jnp.transpose(jnp.where(valid, slotv, SLOTS))[0]
        sg_ref[0, 0, :] = jnp.transpose(jnp.where(valid, slotv, 0))[0]
        g_ref[0, :, :] = jnp.broadcast_to(jnp.where(valid, mk, 0.0), (TBLK, 16))

    @pl.when(b == NB - 1)
    def _():
        cnts = counts_ref[...]
        ps = psum_ref[...]
        balance = jnp.sum(ps / NTOK * (cnts / NTOK)) * E
        imp = jnp.sum(ps * ps) / E
        aux_ref[...] = jnp.reshape(balance + imp, (1, 1))


def _run_router(x, Wr, br):
    br2 = br.reshape(1, E)
    islot = jax.ShapeDtypeStruct((NB, 1, TBLK), jnp.int32)
    fgate = jax.ShapeDtypeStruct((NB, TBLK, 16), jnp.float32)
    out_shape = [
        islot, islot, islot, islot,
        fgate, fgate,
        jax.ShapeDtypeStruct((NTOK, DIN), jnp.bfloat16),
        jax.ShapeDtypeStruct((1, E), jnp.float32),
        jax.ShapeDtypeStruct((1, 1), jnp.float32),
    ]
    slot_spec = pl.BlockSpec((1, 1, TBLK), lambda b: (b, 0, 0))
    gate_spec = pl.BlockSpec((1, TBLK, 16), lambda b: (b, 0, 0))
    return pl.pallas_call(
        _router_body,
        grid=(NB,),
        in_specs=[
            pl.BlockSpec((TBLK, DIN), lambda b: (b, 0)),
            pl.BlockSpec((DIN, E), lambda b: (0, 0)),
            pl.BlockSpec((1, E), lambda b: (0, 0)),
        ],
        out_specs=[
            slot_spec, slot_spec, slot_spec, slot_spec,
            gate_spec, gate_spec,
            pl.BlockSpec((TBLK, DIN), lambda b: (b, 0)),
            pl.BlockSpec((1, E), lambda b: (0, 0)),
            pl.BlockSpec((1, 1), lambda b: (0, 0)),
        ],
        out_shape=out_shape,
        scratch_shapes=[pltpu.VMEM((1, E), jnp.float32)],
    )(x, Wr, br2)


DPK = DIN // 2


def _dispatch(xp, ss0, ss1):
    mesh = plsc.VectorSubcoreMesh(core_axis_name="c", subcore_axis_name="s")

    @functools.partial(
        pl.kernel,
        mesh=mesh,
        out_type=jax.ShapeDtypeStruct((XG_ROWS, DPK), jnp.int32),
        scratch_types=[
            pltpu.VMEM((CHUNK, DPK), jnp.int32),
            pltpu.VMEM((CHUNK,), jnp.int32),
            pltpu.VMEM((CHUNK,), jnp.int32),
            pltpu.SemaphoreType.DMA,
        ],
    )
    def body(x_hbm, ss0_hbm, ss1_hbm, xg_hbm, xbuf, idx0, idx1, sem):
        wid = lax.axis_index("s") * 2 + lax.axis_index("c")
        for ci in range(NCHUNK):
            t0 = wid * TOK_PER_W + ci * CHUNK
            pltpu.sync_copy(x_hbm.at[pl.ds(t0, CHUNK)], xbuf)
            pltpu.sync_copy(ss0_hbm.at[pl.ds(t0, CHUNK)], idx0)
            pltpu.sync_copy(ss1_hbm.at[pl.ds(t0, CHUNK)], idx1)
            pltpu.async_copy(xbuf, xg_hbm.at[idx0], sem).wait()
            pltpu.async_copy(xbuf, xg_hbm.at[idx1], sem).wait()

    return body(xp, ss0, ss1)


def _ffn_body(cnt_ref, xg_ref, w1_ref, b1_ref, w2_ref, b2_ref, yg_ref):
    e = pl.program_id(0)
    cnt = cnt_ref[e]
    rowmask = lax.broadcasted_iota(jnp.int32, (CAP, 1), 0) < cnt
    xb = jnp.where(rowmask, xg_ref[...].astype(jnp.float32), 0.0)
    h = jnp.dot(xb, w1_ref[0], preferred_element_type=jnp.float32) + b1_ref[0]
    h = 0.5 * h * (1.0 + lax.erf(h * 0.7071067811865476))
    y = jnp.dot(h, w2_ref[0], preferred_element_type=jnp.float32) + b2_ref[0]
    yg_ref[...] = y


def _run_ffn(counts_i, xg, W1, b1, W2, b2):
    grid_spec = pltpu.PrefetchScalarGridSpec(
        num_scalar_prefetch=1,
        grid=(E,),
        in_specs=[
            pl.BlockSpec((CAP, DIN), lambda e, c: (e, 0)),
            pl.BlockSpec((1, DIN, DHID), lambda e, c: (e, 0, 0)),
            pl.BlockSpec((1, 1, DHID), lambda e, c: (e, 0, 0)),
            pl.BlockSpec((1, DHID, DOUT), lambda e, c: (e, 0, 0)),
            pl.BlockSpec((1, 1, DOUT), lambda e, c: (e, 0, 0)),
        ],
        out_specs=pl.BlockSpec((CAP, DOUT), lambda e, c: (e, 0)),
    )
    return pl.pallas_call(
        _ffn_body,
        grid_spec=grid_spec,
        out_shape=jax.ShapeDtypeStruct((SLOTS, DOUT), jnp.float32),
    )(counts_i, xg, W1, b1.reshape(E, 1, DHID), W2, b2.reshape(E, 1, DOUT))


CCH = 8
NCC = TOK_PER_W // CCH


def _combine(yg, sg0, sg1, g0, g1):
    mesh = plsc.VectorSubcoreMesh(core_axis_name="c", subcore_axis_name="s")
    nvec = DOUT // 16

    @functools.partial(
        pl.kernel,
        mesh=mesh,
        out_type=jax.ShapeDtypeStruct((NTOK, DOUT), jnp.float32),
        scratch_types=[
            pltpu.VMEM((2, CCH, DOUT), jnp.float32),
            pltpu.VMEM((2, CCH, DOUT), jnp.float32),
            pltpu.VMEM((2, CCH, DOUT), jnp.float32),
            pltpu.VMEM((TOK_PER_W,), jnp.int32),
            pltpu.VMEM((TOK_PER_W,), jnp.int32),
            pltpu.VMEM((TOK_PER_W, 16), jnp.float32),
            pltpu.VMEM((TOK_PER_W, 16), jnp.float32),
            pltpu.SemaphoreType.DMA,
            pltpu.SemaphoreType.DMA,
            pltpu.SemaphoreType.DMA,
            pltpu.SemaphoreType.DMA,
            pltpu.SemaphoreType.DMA,
            pltpu.SemaphoreType.DMA,
        ],
    )
    def body(yg_hbm, sg0_hbm, sg1_hbm, g0_hbm, g1_hbm, out_hbm,
             b0, b1, ob, idx0, idx1, g0a, g1a,
             gsA, gsB, g1sA, g1sB, stA, stB):
        wid = lax.axis_index("s") * 2 + lax.axis_index("c")
        base = wid * TOK_PER_W
        gs = (gsA, gsB)
        g1s = (g1sA, g1sB)
        sts = (stA, stB)
        pltpu.sync_copy(sg0_hbm.at[pl.ds(base, TOK_PER_W)], idx0)
        pltpu.sync_copy(sg1_hbm.at[pl.ds(base, TOK_PER_W)], idx1)
        pltpu.sync_copy(g0_hbm.at[pl.ds(base, TOK_PER_W)], g0a)
        pltpu.sync_copy(g1_hbm.at[pl.ds(base, TOK_PER_W)], g1a)

        def gat(c, s):
            off = c * CCH
            pltpu.async_copy(yg_hbm.at[idx0.at[pl.ds(off, CCH)]], b0.at[s], gs[s])
            pltpu.async_copy(yg_hbm.at[idx1.at[pl.ds(off, CCH)]], b1.at[s], g1s[s])

        def wait_g(s):
            pltpu.make_async_copy(yg_hbm.at[idx0.at[pl.ds(0, CCH)]], b0.at[s], gs[s]).wait()
            pltpu.make_async_copy(yg_hbm.at[idx1.at[pl.ds(0, CCH)]], b1.at[s], g1s[s]).wait()

        gat(0, 0)
        gat(1, 1)

        def step(i, _):
            for s in (0, 1):
                c = 2 * i + s
                wait_g(s)

                @pl.when(c >= 2)
                def _():
                    pltpu.make_async_copy(
                        ob.at[s], out_hbm.at[pl.ds(base, CCH)], sts[s]).wait()

                def row(j, _):
                    ga = g0a[c * CCH + j, :]
                    gb = g1a[c * CCH + j, :]
                    for v in range(nvec):
                        sl = pl.ds(v * 16, 16)
                        ob[s, j, sl] = ga * b0[s, j, sl] + gb * b1[s, j, sl]
                    return 0

                lax.fori_loop(0, CCH, row, 0)
                pltpu.async_copy(
                    ob.at[s], out_hbm.at[pl.ds(base + c * CCH, CCH)], sts[s])

                @pl.when(c + 2 < NCC)
                def _():
                    gat(c + 2, s)
            return 0

        lax.fori_loop(0, NCC // 2, step, 0)
        for s in (0, 1):
            pltpu.make_async_copy(
                ob.at[s], out_hbm.at[pl.ds(base, CCH)], sts[s]).wait()

    return body(yg, sg0, sg1, g0, g1)


def kernel(x, Wr, br, W1, b1, W2, b2):
    (ss0, ss1, sg0, sg1, g0, g1, xbf, counts, aux) = _run_router(x, Wr, br)
    counts_i = counts[0].astype(jnp.int32)
    xp = lax.bitcast_convert_type(xbf.reshape(NTOK, DPK, 2), jnp.int32)
    xg_p = _dispatch(xp, ss0.reshape(NTOK), ss1.reshape(NTOK))
    xg = lax.bitcast_convert_type(xg_p, jnp.bfloat16).reshape(XG_ROWS, DIN)
    yg = _run_ffn(counts_i, xg, W1, b1, W2, b2)
    final = _combine(yg, sg0.reshape(NTOK), sg1.reshape(NTOK),
                     g0.reshape(NTOK, 16), g1.reshape(NTOK, 16))
    return final, aux[0, 0]


# speedup vs baseline: 3.0932x; 3.0932x over previous
import functools

import jax
import jax.numpy as jnp
from jax import lax
from jax.experimental import pallas as pl
from jax.experimental.pallas import tpu as pltpu
from jax.experimental.pallas import tpu_sc as plsc

E = 64
K = 2
CAP = 512
DIN = 1024
DHID = 1024
DOUT = 1024
NTOK = 8192
TBLK = 1024
NB = NTOK // TBLK
SLOTS = E * CAP
XG_ROWS = SLOTS + CAP

NW = 32
TOK_PER_W = NTOK // NW
CHUNK = 32
NCHUNK = TOK_PER_W // CHUNK


def _router_body(x_ref, wr_ref, br_ref, tri_ref,
                 ss0_ref, ss1_ref, sg0_ref, sg1_ref, g0_ref, g1_ref,
                 counts_ref, aux_ref, psum_ref):
    b = pl.program_id(0)

    @pl.when(b == 0)
    def _():
        counts_ref[...] = jnp.zeros_like(counts_ref)
        psum_ref[...] = jnp.zeros_like(psum_ref)

    x = x_ref[...]
    logits = jnp.dot(x, wr_ref[...], preferred_element_type=jnp.float32)
    logits = logits + br_ref[...]
    m = jnp.max(logits, axis=1, keepdims=True)
    ex = jnp.exp(logits - m)
    probs = ex / jnp.sum(ex, axis=1, keepdims=True)
    psum_ref[...] += jnp.sum(probs, axis=0, keepdims=True)

    lane = lax.broadcasted_iota(jnp.int32, (TBLK, E), 1)
    m1 = jnp.max(probs, axis=1, keepdims=True)
    idx1 = jnp.min(jnp.where(probs == m1, lane, E), axis=1, keepdims=True)
    pm = jnp.where(lane == idx1, -1.0, probs)
    m2 = jnp.max(pm, axis=1, keepdims=True)
    idx2 = jnp.min(jnp.where(pm == m2, lane, E), axis=1, keepdims=True)

    oh = ((lane == idx1) | (lane == idx2)).astype(jnp.float32)
    base = counts_ref[...]
    csum_ex = jnp.dot(tri_ref[...], oh, preferred_element_type=jnp.float32)
    posf = csum_ex + base
    counts_ref[...] = base + jnp.sum(oh, axis=0, keepdims=True)

    for idxk, mk, ss_ref, sg_ref, g_ref in (
            (idx1, m1, ss0_ref, sg0_ref, g0_ref),
            (idx2, m2, ss1_ref, sg1_ref, g1_ref)):
        pk = jnp.sum(jnp.where(lane == idxk, posf, 0.0),
                     axis=1, keepdims=True).astype(jnp.int32)
        slotv = idxk * CAP + pk
        valid = pk < CAP
        ss_ref[0, 0, :] = jnp.transpose(jnp.where(valid, slotv, SLOTS))[0]
        sg_ref[0, 0, :] = jnp.transpose(jnp.where(valid, slotv, 0))[0]
        g_ref[0, :, :] = jnp.broadcast_to(jnp.where(valid, mk, 0.0), (TBLK, 16))

    @pl.when(b == NB - 1)
    def _():
        cnts = counts_ref[...]
        ps = psum_ref[...]
        balance = jnp.sum(ps / NTOK * (cnts / NTOK)) * E
        imp = jnp.sum(ps * ps) / E
        aux_ref[...] = jnp.reshape(balance + imp, (1, 1))


def _run_router(x, Wr, br):
    br2 = br.reshape(1, E)
    tri = jnp.tril(jnp.ones((TBLK, TBLK), jnp.float32), -1)
    islot = jax.ShapeDtypeStruct((NB, 1, TBLK), jnp.int32)
    fgate = jax.ShapeDtypeStruct((NB, TBLK, 16), jnp.float32)
    out_shape = [
        islot, islot, islot, islot,
        fgate, fgate,
        jax.ShapeDtypeStruct((1, E), jnp.float32),
        jax.ShapeDtypeStruct((1, 1), jnp.float32),
    ]
    slot_spec = pl.BlockSpec((1, 1, TBLK), lambda b: (b, 0, 0))
    gate_spec = pl.BlockSpec((1, TBLK, 16), lambda b: (b, 0, 0))
    return pl.pallas_call(
        _router_body,
        grid=(NB,),
        in_specs=[
            pl.BlockSpec((TBLK, DIN), lambda b: (b, 0)),
            pl.BlockSpec((DIN, E), lambda b: (0, 0)),
            pl.BlockSpec((1, E), lambda b: (0, 0)),
            pl.BlockSpec((TBLK, TBLK), lambda b: (0, 0)),
        ],
        out_specs=[
            slot_spec, slot_spec, slot_spec, slot_spec,
            gate_spec, gate_spec,
            pl.BlockSpec((1, E), lambda b: (0, 0)),
            pl.BlockSpec((1, 1), lambda b: (0, 0)),
        ],
        out_shape=out_shape,
        scratch_shapes=[pltpu.VMEM((1, E), jnp.float32)],
    )(x, Wr, br2, tri)


def _dispatch(x, ss0, ss1):
    mesh = plsc.VectorSubcoreMesh(core_axis_name="c", subcore_axis_name="s")

    @functools.partial(
        pl.kernel,
        mesh=mesh,
        out_type=jax.ShapeDtypeStruct((XG_ROWS, DIN), jnp.float32),
        scratch_types=[
            pltpu.VMEM((2, CHUNK, DIN), jnp.float32),
            pltpu.VMEM((NCHUNK, CHUNK), jnp.int32),
            pltpu.VMEM((NCHUNK, CHUNK), jnp.int32),
            pltpu.SemaphoreType.DMA,
            pltpu.SemaphoreType.DMA,
            pltpu.SemaphoreType.DMA,
            pltpu.SemaphoreType.DMA,
        ],
    )
    def body(x_hbm, ss0_hbm, ss1_hbm, xg_hbm, xbuf, idx0, idx1,
             ldA, ldB, scA, scB):
        wid = lax.axis_index("s") * 2 + lax.axis_index("c")
        base = wid * TOK_PER_W
        lds = (ldA, ldB)
        scs = (scA, scB)
        pltpu.sync_copy(ss0_hbm.at[wid], idx0)
        pltpu.sync_copy(ss1_hbm.at[wid], idx1)

        def load(ci, s):
            pltpu.async_copy(x_hbm.at[pl.ds(base + ci * CHUNK, CHUNK)],
                             xbuf.at[s], lds[s])

        def wait_scat(ci, s):
            pltpu.make_async_copy(xbuf.at[s], xg_hbm.at[idx0.at[ci]],
                                  scs[s]).wait()
            pltpu.make_async_copy(xbuf.at[s], xg_hbm.at[idx1.at[ci]],
                                  scs[s]).wait()

        load(0, 0)
        load(1, 1)
        for ci in range(NCHUNK):
            s = ci % 2
            pltpu.make_async_copy(x_hbm.at[pl.ds(base, CHUNK)],
                                  xbuf.at[s], lds[s]).wait()
            pltpu.async_copy(xbuf.at[s], xg_hbm.at[idx0.at[ci]], scs[s])
            pltpu.async_copy(xbuf.at[s], xg_hbm.at[idx1.at[ci]], scs[s])
            if ci + 2 < NCHUNK:
                wait_scat(ci, s)
                load(ci + 2, s)
        wait_scat(NCHUNK - 2, 0 if (NCHUNK - 2) % 2 == 0 else 1)
        wait_scat(NCHUNK - 1, 0 if (NCHUNK - 1) % 2 == 0 else 1)

    return body(x, ss0.reshape(NW, NCHUNK, CHUNK), ss1.reshape(NW, NCHUNK, CHUNK))


def _ffn_body(cnt_ref, xg_ref, w1_ref, b1_ref, w2_ref, b2_ref, yg_ref):
    e = pl.program_id(0)
    cnt = cnt_ref[e]
    rowmask = lax.broadcasted_iota(jnp.int32, (CAP, 1), 0) < cnt
    xb = jnp.where(rowmask, xg_ref[...], 0.0)
    h = jnp.dot(xb, w1_ref[0], preferred_element_type=jnp.float32) + b1_ref[0]
    h = 0.5 * h * (1.0 + lax.erf(h * 0.7071067811865476))
    y = jnp.dot(h, w2_ref[0], preferred_element_type=jnp.float32) + b2_ref[0]
    yg_ref[...] = y


def _run_ffn(counts_i, xg, W1, b1, W2, b2):
    grid_spec = pltpu.PrefetchScalarGridSpec(
        num_scalar_prefetch=1,
        grid=(E,),
        in_specs=[
            pl.BlockSpec((CAP, DIN), lambda e, c: (e, 0)),
            pl.BlockSpec((1, DIN, DHID), lambda e, c: (e, 0, 0)),
            pl.BlockSpec((1, 1, DHID), lambda e, c: (e, 0, 0)),
            pl.BlockSpec((1, DHID, DOUT), lambda e, c: (e, 0, 0)),
            pl.BlockSpec((1, 1, DOUT), lambda e, c: (e, 0, 0)),
        ],
        out_specs=pl.BlockSpec((CAP, DOUT), lambda e, c: (e, 0)),
    )
    return pl.pallas_call(
        _ffn_body,
        grid_spec=grid_spec,
        out_shape=jax.ShapeDtypeStruct((SLOTS, DOUT), jnp.float32),
    )(counts_i, xg, W1, b1.reshape(E, 1, DHID), W2, b2.reshape(E, 1, DOUT))


CCH = 8
NCC = TOK_PER_W // CCH


def _combine(yg, sg0, sg1, g0, g1):
    mesh = plsc.VectorSubcoreMesh(core_axis_name="c", subcore_axis_name="s")
    nvec = DOUT // 16

    @functools.partial(
        pl.kernel,
        mesh=mesh,
        out_type=jax.ShapeDtypeStruct((NTOK, DOUT), jnp.float32),
        scratch_types=[
            pltpu.VMEM((2, CCH, DOUT), jnp.float32),
            pltpu.VMEM((2, CCH, DOUT), jnp.float32),
            pltpu.VMEM((2, CCH, DOUT), jnp.float32),
            pltpu.VMEM((TOK_PER_W,), jnp.int32),
            pltpu.VMEM((TOK_PER_W,), jnp.int32),
            pltpu.VMEM((TOK_PER_W, 16), jnp.float32),
            pltpu.VMEM((TOK_PER_W, 16), jnp.float32),
            pltpu.SemaphoreType.DMA,
            pltpu.SemaphoreType.DMA,
            pltpu.SemaphoreType.DMA,
            pltpu.SemaphoreType.DMA,
            pltpu.SemaphoreType.DMA,
            pltpu.SemaphoreType.DMA,
        ],
    )
    def body(yg_hbm, sg0_hbm, sg1_hbm, g0_hbm, g1_hbm, out_hbm,
             b0, b1, ob, idx0, idx1, g0a, g1a,
             gsA, gsB, g1sA, g1sB, stA, stB):
        wid = lax.axis_index("s") * 2 + lax.axis_index("c")
        base = wid * TOK_PER_W
        gs = (gsA, gsB)
        g1s = (g1sA, g1sB)
        sts = (stA, stB)
        pltpu.sync_copy(sg0_hbm.at[pl.ds(base, TOK_PER_W)], idx0)
        pltpu.sync_copy(sg1_hbm.at[pl.ds(base, TOK_PER_W)], idx1)
        pltpu.sync_copy(g0_hbm.at[pl.ds(base, TOK_PER_W)], g0a)
        pltpu.sync_copy(g1_hbm.at[pl.ds(base, TOK_PER_W)], g1a)

        def gat(c, s):
            off = c * CCH
            pltpu.async_copy(yg_hbm.at[idx0.at[pl.ds(off, CCH)]], b0.at[s], gs[s])
            pltpu.async_copy(yg_hbm.at[idx1.at[pl.ds(off, CCH)]], b1.at[s], g1s[s])

        def wait_g(s):
            pltpu.make_async_copy(yg_hbm.at[idx0.at[pl.ds(0, CCH)]], b0.at[s], gs[s]).wait()
            pltpu.make_async_copy(yg_hbm.at[idx1.at[pl.ds(0, CCH)]], b1.at[s], g1s[s]).wait()

        gat(0, 0)
        gat(1, 1)

        def step(i, _):
            for s in (0, 1):
                c = 2 * i + s
                wait_g(s)

                @pl.when(c >= 2)
                def _():
                    pltpu.make_async_copy(
                        ob.at[s], out_hbm.at[pl.ds(base, CCH)], sts[s]).wait()

                def row(j, _):
                    ga = g0a[c * CCH + j, :]
                    gb = g1a[c * CCH + j, :]
                    for v in range(nvec):
                        sl = pl.ds(v * 16, 16)
                        ob[s, j, sl] = ga * b0[s, j, sl] + gb * b1[s, j, sl]
                    return 0

                lax.fori_loop(0, CCH, row, 0)
                pltpu.async_copy(
                    ob.at[s], out_hbm.at[pl.ds(base + c * CCH, CCH)], sts[s])

                @pl.when(c + 2 < NCC)
                def _():
                    gat(c + 2, s)
            return 0

        lax.fori_loop(0, NCC // 2, step, 0)
        for s in (0, 1):
            pltpu.make_async_copy(
                ob.at[s], out_hbm.at[pl.ds(base, CCH)], sts[s]).wait()

    return body(yg, sg0, sg1, g0, g1)


def kernel(x, Wr, br, W1, b1, W2, b2):
    (ss0, ss1, sg0, sg1, g0, g1, counts, aux) = _run_router(x, Wr, br)
    counts_i = counts[0].astype(jnp.int32)
    xg = _dispatch(x, ss0.reshape(NTOK), ss1.reshape(NTOK))
    yg = _run_ffn(counts_i, xg, W1, b1, W2, b2)
    final = _combine(yg, sg0.reshape(NTOK), sg1.reshape(NTOK),
                     g0.reshape(NTOK, 16), g1.reshape(NTOK, 16))
    return final, aux[0, 0]


# speedup vs baseline: 3.1143x; 1.0068x over previous
import functools

import jax
import jax.numpy as jnp
from jax import lax
from jax.experimental import pallas as pl
from jax.experimental.pallas import tpu as pltpu
from jax.experimental.pallas import tpu_sc as plsc

E = 64
K = 2
CAP = 512
DIN = 1024
DHID = 1024
DOUT = 1024
NTOK = 8192
TBLK = 512
NB = NTOK // TBLK
SLOTS = E * CAP
XG_ROWS = SLOTS + CAP

NW = 32
TOK_PER_W = NTOK // NW
CHUNK = 32
NCHUNK = TOK_PER_W // CHUNK


def _router_body(x_ref, wr_ref, br_ref, tri_ref,
                 ss0_ref, ss1_ref, sg0_ref, sg1_ref, g0_ref, g1_ref,
                 counts_ref, aux_ref, psum_ref):
    b = pl.program_id(0)

    @pl.when(b == 0)
    def _():
        counts_ref[...] = jnp.zeros_like(counts_ref)
        psum_ref[...] = jnp.zeros_like(psum_ref)

    x = x_ref[...]
    logits = jnp.dot(x, wr_ref[...], preferred_element_type=jnp.float32)
    logits = logits + br_ref[...]
    m = jnp.max(logits, axis=1, keepdims=True)
    ex = jnp.exp(logits - m)
    probs = ex / jnp.sum(ex, axis=1, keepdims=True)
    psum_ref[...] += jnp.sum(probs, axis=0, keepdims=True)

    lane = lax.broadcasted_iota(jnp.int32, (TBLK, E), 1)
    m1 = jnp.max(probs, axis=1, keepdims=True)
    idx1 = jnp.min(jnp.where(probs == m1, lane, E), axis=1, keepdims=True)
    pm = jnp.where(lane == idx1, -1.0, probs)
    m2 = jnp.max(pm, axis=1, keepdims=True)
    idx2 = jnp.min(jnp.where(pm == m2, lane, E), axis=1, keepdims=True)

    oh = ((lane == idx1) | (lane == idx2)).astype(jnp.float32)
    base = counts_ref[...]
    csum_ex = jnp.dot(tri_ref[...], oh, preferred_element_type=jnp.float32)
    posf = csum_ex + base
    counts_ref[...] = base + jnp.sum(oh, axis=0, keepdims=True)

    for idxk, mk, ss_ref, sg_ref, g_ref in (
            (idx1, m1, ss0_ref, sg0_ref, g0_ref),
            (idx2, m2, ss1_ref, sg1_ref, g1_ref)):
        pk = jnp.sum(jnp.where(lane == idxk, posf, 0.0),
                     axis=1, keepdims=True).astype(jnp.int32)
        slotv = idxk * CAP + pk
        valid = pk < CAP
        ss_ref[0, 0, :] = jnp.transpose(jnp.where(valid, slotv, SLOTS))[0]
        sg_ref[0, 0, :] = jnp.transpose(jnp.where(valid, slotv, 0))[0]
        g_ref[0, :, :] = jnp.broadcast_to(jnp.where(valid, mk, 0.0), (TBLK, 16))

    @pl.when(b == NB - 1)
    def _():
        cnts = counts_ref[...]
        ps = psum_ref[...]
        balance = jnp.sum(ps / NTOK * (cnts / NTOK)) * E
        imp = jnp.sum(ps * ps) / E
        aux_ref[...] = jnp.reshape(balance + imp, (1, 1))


def _run_router(x, Wr, br):
    br2 = br.reshape(1, E)
    tri = jnp.tril(jnp.ones((TBLK, TBLK), jnp.float32), -1)
    islot = jax.ShapeDtypeStruct((NB, 1, TBLK), jnp.int32)
    fgate = jax.ShapeDtypeStruct((NB, TBLK, 16), jnp.float32)
    out_shape = [
        islot, islot, islot, islot,
        fgate, fgate,
        jax.ShapeDtypeStruct((1, E), jnp.float32),
        jax.ShapeDtypeStruct((1, 1), jnp.float32),
    ]
    slot_spec = pl.BlockSpec((1, 1, TBLK), lambda b: (b, 0, 0))
    gate_spec = pl.BlockSpec((1, TBLK, 16), lambda b: (b, 0, 0))
    return pl.pallas_call(
        _router_body,
        grid=(NB,),
        in_specs=[
            pl.BlockSpec((TBLK, DIN), lambda b: (b, 0)),
            pl.BlockSpec((DIN, E), lambda b: (0, 0)),
            pl.BlockSpec((1, E), lambda b: (0, 0)),
            pl.BlockSpec((TBLK, TBLK), lambda b: (0, 0)),
        ],
        out_specs=[
            slot_spec, slot_spec, slot_spec, slot_spec,
            gate_spec, gate_spec,
            pl.BlockSpec((1, E), lambda b: (0, 0)),
            pl.BlockSpec((1, 1), lambda b: (0, 0)),
        ],
        out_shape=out_shape,
        scratch_shapes=[pltpu.VMEM((1, E), jnp.float32)],
    )(x, Wr, br2, tri)


def _dispatch(x, ss0, ss1):
    mesh = plsc.VectorSubcoreMesh(core_axis_name="c", subcore_axis_name="s")

    @functools.partial(
        pl.kernel,
        mesh=mesh,
        out_type=jax.ShapeDtypeStruct((XG_ROWS, DIN), jnp.float32),
        scratch_types=[
            pltpu.VMEM((2, CHUNK, DIN), jnp.float32),
            pltpu.VMEM((NCHUNK, CHUNK), jnp.int32),
            pltpu.VMEM((NCHUNK, CHUNK), jnp.int32),
            pltpu.SemaphoreType.DMA,
            pltpu.SemaphoreType.DMA,
            pltpu.SemaphoreType.DMA,
            pltpu.SemaphoreType.DMA,
        ],
    )
    def body(x_hbm, ss0_hbm, ss1_hbm, xg_hbm, xbuf, idx0, idx1,
             ldA, ldB, scA, scB):
        wid = lax.axis_index("s") * 2 + lax.axis_index("c")
        base = wid * TOK_PER_W
        lds = (ldA, ldB)
        scs = (scA, scB)
        pltpu.sync_copy(ss0_hbm.at[wid], idx0)
        pltpu.sync_copy(ss1_hbm.at[wid], idx1)

        def load(ci, s):
            pltpu.async_copy(x_hbm.at[pl.ds(base + ci * CHUNK, CHUNK)],
                             xbuf.at[s], lds[s])

        def wait_scat(ci, s):
            pltpu.make_async_copy(xbuf.at[s], xg_hbm.at[idx0.at[ci]],
                                  scs[s]).wait()
            pltpu.make_async_copy(xbuf.at[s], xg_hbm.at[idx1.at[ci]],
                                  scs[s]).wait()

        load(0, 0)
        load(1, 1)
        for ci in range(NCHUNK):
            s = ci % 2
            pltpu.make_async_copy(x_hbm.at[pl.ds(base, CHUNK)],
                                  xbuf.at[s], lds[s]).wait()
            pltpu.async_copy(xbuf.at[s], xg_hbm.at[idx0.at[ci]], scs[s])
            pltpu.async_copy(xbuf.at[s], xg_hbm.at[idx1.at[ci]], scs[s])
            if ci + 2 < NCHUNK:
                wait_scat(ci, s)
                load(ci + 2, s)
        wait_scat(NCHUNK - 2, 0 if (NCHUNK - 2) % 2 == 0 else 1)
        wait_scat(NCHUNK - 1, 0 if (NCHUNK - 1) % 2 == 0 else 1)

    return body(x, ss0.reshape(NW, NCHUNK, CHUNK), ss1.reshape(NW, NCHUNK, CHUNK))


def _ffn_body(cnt_ref, xg_ref, w1_ref, b1_ref, w2_ref, b2_ref, yg_ref):
    e = pl.program_id(0)
    cnt = cnt_ref[e]
    rowmask = lax.broadcasted_iota(jnp.int32, (CAP, 1), 0) < cnt
    xb = jnp.where(rowmask, xg_ref[...], 0.0)
    h = jnp.dot(xb, w1_ref[0], preferred_element_type=jnp.float32) + b1_ref[0]
    h = 0.5 * h * (1.0 + lax.erf(h * 0.7071067811865476))
    y = jnp.dot(h, w2_ref[0], preferred_element_type=jnp.float32) + b2_ref[0]
    yg_ref[...] = y


def _run_ffn(counts_i, xg, W1, b1, W2, b2):
    grid_spec = pltpu.PrefetchScalarGridSpec(
        num_scalar_prefetch=1,
        grid=(E,),
        in_specs=[
            pl.BlockSpec((CAP, DIN), lambda e, c: (e, 0)),
            pl.BlockSpec((1, DIN, DHID), lambda e, c: (e, 0, 0)),
            pl.BlockSpec((1, 1, DHID), lambda e, c: (e, 0, 0)),
            pl.BlockSpec((1, DHID, DOUT), lambda e, c: (e, 0, 0)),
            pl.BlockSpec((1, 1, DOUT), lambda e, c: (e, 0, 0)),
        ],
        out_specs=pl.BlockSpec((CAP, DOUT), lambda e, c: (e, 0)),
    )
    return pl.pallas_call(
        _ffn_body,
        grid_spec=grid_spec,
        out_shape=jax.ShapeDtypeStruct((SLOTS, DOUT), jnp.float32),
    )(counts_i, xg, W1, b1.reshape(E, 1, DHID), W2, b2.reshape(E, 1, DOUT))


CCH = 8
NCC = TOK_PER_W // CCH


def _combine(yg, sg0, sg1, g0, g1):
    mesh = plsc.VectorSubcoreMesh(core_axis_name="c", subcore_axis_name="s")
    nvec = DOUT // 16

    @functools.partial(
        pl.kernel,
        mesh=mesh,
        out_type=jax.ShapeDtypeStruct((NTOK, DOUT), jnp.float32),
        scratch_types=[
            pltpu.VMEM((2, CCH, DOUT), jnp.float32),
            pltpu.VMEM((2, CCH, DOUT), jnp.float32),
            pltpu.VMEM((2, CCH, DOUT), jnp.float32),
            pltpu.VMEM((TOK_PER_W,), jnp.int32),
            pltpu.VMEM((TOK_PER_W,), jnp.int32),
            pltpu.VMEM((TOK_PER_W, 16), jnp.float32),
            pltpu.VMEM((TOK_PER_W, 16), jnp.float32),
            pltpu.SemaphoreType.DMA,
            pltpu.SemaphoreType.DMA,
            pltpu.SemaphoreType.DMA,
            pltpu.SemaphoreType.DMA,
            pltpu.SemaphoreType.DMA,
            pltpu.SemaphoreType.DMA,
        ],
    )
    def body(yg_hbm, sg0_hbm, sg1_hbm, g0_hbm, g1_hbm, out_hbm,
             b0, b1, ob, idx0, idx1, g0a, g1a,
             gsA, gsB, g1sA, g1sB, stA, stB):
        wid = lax.axis_index("s") * 2 + lax.axis_index("c")
        base = wid * TOK_PER_W
        gs = (gsA, gsB)
        g1s = (g1sA, g1sB)
        sts = (stA, stB)
        pltpu.sync_copy(sg0_hbm.at[pl.ds(base, TOK_PER_W)], idx0)
        pltpu.sync_copy(sg1_hbm.at[pl.ds(base, TOK_PER_W)], idx1)
        pltpu.sync_copy(g0_hbm.at[pl.ds(base, TOK_PER_W)], g0a)
        pltpu.sync_copy(g1_hbm.at[pl.ds(base, TOK_PER_W)], g1a)

        def gat(c, s):
            off = c * CCH
            pltpu.async_copy(yg_hbm.at[idx0.at[pl.ds(off, CCH)]], b0.at[s], gs[s])
            pltpu.async_copy(yg_hbm.at[idx1.at[pl.ds(off, CCH)]], b1.at[s], g1s[s])

        def wait_g(s):
            pltpu.make_async_copy(yg_hbm.at[idx0.at[pl.ds(0, CCH)]], b0.at[s], gs[s]).wait()
            pltpu.make_async_copy(yg_hbm.at[idx1.at[pl.ds(0, CCH)]], b1.at[s], g1s[s]).wait()

        gat(0, 0)
        gat(1, 1)

        def step(i, _):
            for s in (0, 1):
                c = 2 * i + s
                wait_g(s)

                @pl.when(c >= 2)
                def _():
                    pltpu.make_async_copy(
                        ob.at[s], out_hbm.at[pl.ds(base, CCH)], sts[s]).wait()

                def row(j, _):
                    ga = g0a[c * CCH + j, :]
                    gb = g1a[c * CCH + j, :]
                    for v in range(nvec):
                        sl = pl.ds(v * 16, 16)
                        ob[s, j, sl] = ga * b0[s, j, sl] + gb * b1[s, j, sl]
                    return 0

                lax.fori_loop(0, CCH, row, 0)
                pltpu.async_copy(
                    ob.at[s], out_hbm.at[pl.ds(base + c * CCH, CCH)], sts[s])

                @pl.when(c + 2 < NCC)
                def _():
                    gat(c + 2, s)
            return 0

        lax.fori_loop(0, NCC // 2, step, 0)
        for s in (0, 1):
            pltpu.make_async_copy(
                ob.at[s], out_hbm.at[pl.ds(base, CCH)], sts[s]).wait()

    return body(yg, sg0, sg1, g0, g1)


def kernel(x, Wr, br, W1, b1, W2, b2):
    (ss0, ss1, sg0, sg1, g0, g1, counts, aux) = _run_router(x, Wr, br)
    counts_i = counts[0].astype(jnp.int32)
    xg = _dispatch(x, ss0.reshape(NTOK), ss1.reshape(NTOK))
    yg = _run_ffn(counts_i, xg, W1, b1, W2, b2)
    final = _combine(yg, sg0.reshape(NTOK), sg1.reshape(NTOK),
                     g0.reshape(NTOK, 16), g1.reshape(NTOK, 16))
    return final, aux[0, 0]
